# baseline (device time: 36165 ns/iter reference)
import jax
import jax.numpy as jnp
from jax import lax
from jax.experimental import pallas as pl
from jax.experimental.pallas import tpu as pltpu

N_DEV = 4


def kernel(x, Win0, Wout0, Win1, Wout1, Win2, Wout2):
    m_per, d = x.shape
    h_per = Win0.shape[1]
    M = N_DEV * m_per

    def body(x_ref, win0, wout0, win1, wout1, win2, wout2, out_ref,
             xfull, partial_bf, rs_recv, wi_buf, wo_buf,
             ag_send_sems, ag_recv_sems, rs_send_sems, rs_recv_sems,
             w_sems):
        my = lax.axis_index("i")
        my_rows = pl.ds(my * m_per, m_per)

        w_copies = []
        for l, (wi, wo) in enumerate(
            ((win0, wout0), (win1, wout1), (win2, wout2))
        ):
            ci = pltpu.make_async_copy(wi, wi_buf.at[l], w_sems.at[2 * l])
            co = pltpu.make_async_copy(wo, wo_buf.at[l], w_sems.at[2 * l + 1])
            ci.start()
            co.start()
            w_copies.append((ci, co))

        xfull[my_rows, :] = x_ref[...].astype(jnp.bfloat16)

        barrier_sem = pltpu.get_barrier_semaphore()
        for delta in range(1, N_DEV):
            pl.semaphore_signal(
                barrier_sem, inc=1,
                device_id=(lax.rem(my + delta, N_DEV),),
                device_id_type=pl.DeviceIdType.MESH,
            )
        pl.semaphore_wait(barrier_sem, N_DEV - 1)

        def chunk_partial(rows, wl, wo):
            h = jnp.dot(xfull[rows, :], wl, preferred_element_type=jnp.float32)
            h = jnp.maximum(h, 0.0).astype(jnp.bfloat16)
            return jnp.dot(h, wo, preferred_element_type=jnp.float32)

        for l in range(3):
            ag_sends = []
            for delta in range(1, N_DEV):
                tgt = lax.rem(my + delta, N_DEV)
                rdma = pltpu.make_async_remote_copy(
                    src_ref=xfull.at[my_rows, :],
                    dst_ref=xfull.at[my_rows, :],
                    send_sem=ag_send_sems.at[delta - 1],
                    recv_sem=ag_recv_sems.at[delta - 1],
                    device_id=(tgt,),
                    device_id_type=pl.DeviceIdType.MESH,
                )
                rdma.start()
                ag_sends.append(rdma)

            ci, co = w_copies[l]
            ci.wait()
            co.wait()
            wl = wi_buf[l]
            wo = wo_buf[l]

            own = chunk_partial(my_rows, wl, wo)

            rs_sends = []
            for delta in (1, 3, 2):
                src_dev = lax.rem(my - delta + N_DEV, N_DEV)
                c_rows = pl.ds(src_dev * m_per, m_per)
                recv = pltpu.make_async_remote_copy(
                    src_ref=xfull.at[my_rows, :],
                    dst_ref=xfull.at[c_rows, :],
                    send_sem=ag_send_sems.at[delta - 1],
                    recv_sem=ag_recv_sems.at[delta - 1],
                    device_id=(my,),
                    device_id_type=pl.DeviceIdType.MESH,
                )
                recv.wait_recv()
                partial_bf[c_rows, :] = chunk_partial(c_rows, wl, wo).astype(
                    jnp.bfloat16
                )
                slot = (N_DEV - delta) - 1
                rdma = pltpu.make_async_remote_copy(
                    src_ref=partial_bf.at[c_rows, :],
                    dst_ref=rs_recv.at[slot],
                    send_sem=rs_send_sems.at[slot],
                    recv_sem=rs_recv_sems.at[slot],
                    device_id=(src_dev,),
                    device_id_type=pl.DeviceIdType.MESH,
                )
                rdma.start()
                rs_sends.append(rdma)
            for rdma in ag_sends:
                rdma.wait_send()

            for slot in range(N_DEV - 1):
                recv = pltpu.make_async_remote_copy(
                    src_ref=partial_bf.at[my_rows, :],
                    dst_ref=rs_recv.at[slot],
                    send_sem=rs_send_sems.at[slot],
                    recv_sem=rs_recv_sems.at[slot],
                    device_id=(my,),
                    device_id_type=pl.DeviceIdType.MESH,
                )
                recv.wait_recv()

            res = own
            for j in range(N_DEV - 1):
                res = res + rs_recv[j].astype(jnp.float32)
            for rdma in rs_sends:
                rdma.wait_send()

            if l < 2:
                xfull[my_rows, :] = res.astype(jnp.bfloat16)
            else:
                out_ref[...] = res

    return pl.pallas_call(
        body,
        out_shape=jax.ShapeDtypeStruct((m_per, d), jnp.float32),
        in_specs=[pl.BlockSpec(memory_space=pltpu.VMEM)]
        + [pl.BlockSpec(memory_space=pl.ANY)] * 6,
        out_specs=pl.BlockSpec(memory_space=pltpu.VMEM),
        scratch_shapes=[
            pltpu.VMEM((M, d), jnp.bfloat16),
            pltpu.VMEM((M, d), jnp.bfloat16),
            pltpu.VMEM((N_DEV - 1, m_per, d), jnp.bfloat16),
            pltpu.VMEM((3, d, h_per), jnp.bfloat16),
            pltpu.VMEM((3, h_per, d), jnp.bfloat16),
            pltpu.SemaphoreType.DMA((N_DEV - 1,)),
            pltpu.SemaphoreType.DMA((N_DEV - 1,)),
            pltpu.SemaphoreType.DMA((N_DEV - 1,)),
            pltpu.SemaphoreType.DMA((N_DEV - 1,)),
            pltpu.SemaphoreType.DMA((6,)),
        ],
        compiler_params=pltpu.CompilerParams(collective_id=0),
    )(
        x,
        Win0.astype(jnp.bfloat16), Wout0.astype(jnp.bfloat16),
        Win1.astype(jnp.bfloat16), Wout1.astype(jnp.bfloat16),
        Win2.astype(jnp.bfloat16), Wout2.astype(jnp.bfloat16),
    )


# device time: 36052 ns/iter; 1.0031x vs baseline; 1.0031x over previous
import jax
import jax.numpy as jnp
from jax import lax
from jax.experimental import pallas as pl
from jax.experimental.pallas import tpu as pltpu

N_DEV = 4


def kernel(x, Win0, Wout0, Win1, Wout1, Win2, Wout2):
    m_per, d = x.shape
    M = N_DEV * m_per

    def body(x_ref, win0, wout0, win1, wout1, win2, wout2, out_ref,
             xfull, partial_bf, rs_recv,
             ag_send_sems, ag_recv_sems, rs_send_sems, rs_recv_sems):
        my = lax.axis_index("i")
        my_rows = pl.ds(my * m_per, m_per)
        xfull[my_rows, :] = x_ref[...].astype(jnp.bfloat16)

        barrier_sem = pltpu.get_barrier_semaphore()
        for delta in range(1, N_DEV):
            pl.semaphore_signal(
                barrier_sem, inc=1,
                device_id=(lax.rem(my + delta, N_DEV),),
                device_id_type=pl.DeviceIdType.MESH,
            )
        pl.semaphore_wait(barrier_sem, N_DEV - 1)

        wins = (win0, win1, win2)
        wouts = (wout0, wout1, wout2)

        def chunk_partial(rows, wl, wo):
            h = jnp.dot(xfull[rows, :], wl, preferred_element_type=jnp.float32)
            h = jnp.maximum(h, 0.0).astype(jnp.bfloat16)
            return jnp.dot(h, wo, preferred_element_type=jnp.float32)

        for l in range(3):
            wl = wins[l][...]
            wo = wouts[l][...]

            ag_sends = []
            for delta in range(1, N_DEV):
                tgt = lax.rem(my + delta, N_DEV)
                rdma = pltpu.make_async_remote_copy(
                    src_ref=xfull.at[my_rows, :],
                    dst_ref=xfull.at[my_rows, :],
                    send_sem=ag_send_sems.at[delta - 1],
                    recv_sem=ag_recv_sems.at[delta - 1],
                    device_id=(tgt,),
                    device_id_type=pl.DeviceIdType.MESH,
                )
                rdma.start()
                ag_sends.append(rdma)

            own = chunk_partial(my_rows, wl, wo)

            rs_sends = []
            for delta in (1, 3, 2):
                src_dev = lax.rem(my - delta + N_DEV, N_DEV)
                c_rows = pl.ds(src_dev * m_per, m_per)
                recv = pltpu.make_async_remote_copy(
                    src_ref=xfull.at[my_rows, :],
                    dst_ref=xfull.at[c_rows, :],
                    send_sem=ag_send_sems.at[delta - 1],
                    recv_sem=ag_recv_sems.at[delta - 1],
                    device_id=(my,),
                    device_id_type=pl.DeviceIdType.MESH,
                )
                recv.wait_recv()
                partial_bf[c_rows, :] = chunk_partial(c_rows, wl, wo).astype(
                    jnp.bfloat16
                )
                slot = (N_DEV - delta) - 1
                rdma = pltpu.make_async_remote_copy(
                    src_ref=partial_bf.at[c_rows, :],
                    dst_ref=rs_recv.at[slot],
                    send_sem=rs_send_sems.at[slot],
                    recv_sem=rs_recv_sems.at[slot],
                    device_id=(src_dev,),
                    device_id_type=pl.DeviceIdType.MESH,
                )
                rdma.start()
                rs_sends.append(rdma)
            for rdma in ag_sends:
                rdma.wait_send()

            for slot in range(N_DEV - 1):
                recv = pltpu.make_async_remote_copy(
                    src_ref=partial_bf.at[my_rows, :],
                    dst_ref=rs_recv.at[slot],
                    send_sem=rs_send_sems.at[slot],
                    recv_sem=rs_recv_sems.at[slot],
                    device_id=(my,),
                    device_id_type=pl.DeviceIdType.MESH,
                )
                recv.wait_recv()

            res = own
            for j in range(N_DEV - 1):
                res = res + rs_recv[j].astype(jnp.float32)
            for rdma in rs_sends:
                rdma.wait_send()

            if l < 2:
                xfull[my_rows, :] = res.astype(jnp.bfloat16)
            else:
                out_ref[...] = res

    return pl.pallas_call(
        body,
        out_shape=jax.ShapeDtypeStruct((m_per, d), jnp.float32),
        in_specs=[pl.BlockSpec(memory_space=pltpu.VMEM)] * 7,
        out_specs=pl.BlockSpec(memory_space=pltpu.VMEM),
        scratch_shapes=[
            pltpu.VMEM((M, d), jnp.bfloat16),
            pltpu.VMEM((M, d), jnp.bfloat16),
            pltpu.VMEM((N_DEV - 1, m_per, d), jnp.bfloat16),
            pltpu.SemaphoreType.DMA((N_DEV - 1,)),
            pltpu.SemaphoreType.DMA((N_DEV - 1,)),
            pltpu.SemaphoreType.DMA((N_DEV - 1,)),
            pltpu.SemaphoreType.DMA((N_DEV - 1,)),
        ],
        compiler_params=pltpu.CompilerParams(collective_id=0),
    )(
        x,
        Win0.astype(jnp.bfloat16), Wout0.astype(jnp.bfloat16),
        Win1.astype(jnp.bfloat16), Wout1.astype(jnp.bfloat16),
        Win2.astype(jnp.bfloat16), Wout2.astype(jnp.bfloat16),
    )
